# Initial kernel scaffold; baseline (speedup 1.0000x reference)
import jax
import jax.numpy as jnp
from jax.experimental import pallas as pl


def kernel(x_tokens, tw_row, fc_b2):
    raise NotImplementedError("write your pallas kernel here")


# speedup vs baseline: 16.0314x; 16.0314x over previous
import functools

import jax
import jax.numpy as jnp
from jax import lax
from jax.experimental import pallas as pl
from jax.experimental.pallas import tpu as pltpu

_BB = 128


def _gather_kernel(ids_ref,
                   tw_ref,
                   b_ref,
                   out_ref,
                   *, sub, inv_len):
    L = ids_ref.shape[0]
    R = tw_ref.shape[0]
    G = R // 8
    del sub

    def body(g, acc):
        base = pl.multiple_of(g * 8, 8)
        chunk = tw_ref[pl.ds(base, 8), :]
        ids = ids_ref[...]
        hi = ids >> 7
        lo = ids & 127
        rows = [jnp.broadcast_to(chunk[rr:rr + 1, :], (L, _BB))
                for rr in range(8)]
        for rr in range(8):
            g_val = jnp.take_along_axis(rows[rr], lo, axis=1)
            acc = acc + jnp.where(hi == g * 8 + rr, g_val, 0.0)
        return acc

    total = lax.fori_loop(0, G, body, jnp.zeros((L, _BB), jnp.float32))

    colsum = jnp.sum(total, axis=0, keepdims=True)
    logits = colsum * inv_len + b_ref[0, 0]
    out_ref[...] = jnp.broadcast_to(logits, out_ref.shape)


def kernel(x_tokens, tw_row, fc_b2):
    L, B = x_tokens.shape
    V = tw_row.shape[1]
    assert V % 1024 == 0 and B % _BB == 0 and L % 8 == 0

    tw2 = tw_row.reshape(V // 128, 128)
    bias = fc_b2.reshape(1, 1)
    sub = min(32, L)

    kern = functools.partial(_gather_kernel, sub=sub, inv_len=1.0 / L)
    out = pl.pallas_call(
        kern,
        out_shape=jax.ShapeDtypeStruct((8, B), jnp.float32),
        grid=(B // _BB,),
        in_specs=[
            pl.BlockSpec((L, _BB), lambda b: (0, b)),
            pl.BlockSpec((V // 128, 128), lambda b: (0, 0)),
            pl.BlockSpec(memory_space=pltpu.MemorySpace.SMEM),
        ],
        out_specs=pl.BlockSpec((8, _BB), lambda b: (0, b)),
        compiler_params=pltpu.CompilerParams(
            dimension_semantics=("parallel",),
            vmem_limit_bytes=32 * 1024 * 1024,
        ),
    )(x_tokens.astype(jnp.int32), tw2, bias)

    return out[0, :]


# speedup vs baseline: 48.0870x; 2.9995x over previous
import functools

import jax
import jax.numpy as jnp
from jax import lax
from jax.experimental import pallas as pl
from jax.experimental.pallas import tpu as pltpu

_BB = 128


def _gather_kernel(ids_ref,
                   tw_ref,
                   b_ref,
                   out_ref,
                   *, inv_len):
    L = ids_ref.shape[0]
    G = tw_ref.shape[0] // 8
    n_rg = L // 8

    def body(g, acc):
        base = pl.multiple_of(g * 8, 8)
        chunk = tw_ref[pl.ds(base, 8), :]
        rows = [jnp.broadcast_to(chunk[rr:rr + 1, :], (8, _BB))
                for rr in range(8)]
        parts = []
        for rg in range(n_rg):
            ids = ids_ref[rg * 8:(rg + 1) * 8, :]
            lo = ids & 127
            pr = ids >> 8
            odd = (ids & 128) > 0
            a = acc[rg * 8:(rg + 1) * 8, :]
            for rr in range(8):
                pair = jnp.take_along_axis(rows[rr], lo, axis=1)
                v_even = pltpu.bitcast(pair << 16, jnp.float32)
                v_odd = pltpu.bitcast(pair & jnp.int32(-65536), jnp.float32)
                val = jnp.where(odd, v_odd, v_even)
                a = a + jnp.where(pr == g * 8 + rr, val, 0.0)
            parts.append(a)
        return jnp.concatenate(parts, axis=0)

    total = lax.fori_loop(0, G, body, jnp.zeros((L, _BB), jnp.float32))

    colsum = jnp.sum(total, axis=0, keepdims=True)
    logits = colsum * inv_len + b_ref[0, 0]
    out_ref[...] = jnp.broadcast_to(logits, out_ref.shape)


def _pack_table(tw_row):
    V = tw_row.shape[1]
    tw2 = tw_row.reshape(V // 128, 128)
    b16 = jax.lax.bitcast_convert_type(tw2.astype(jnp.bfloat16),
                                       jnp.uint16).astype(jnp.uint32)
    packed = (b16[0::2, :] | (b16[1::2, :] << 16)).astype(jnp.uint32)
    P = packed.shape[0]
    P8 = (P + 7) // 8 * 8
    packed = jnp.pad(packed, ((0, P8 - P), (0, 0)))
    return jax.lax.bitcast_convert_type(packed, jnp.int32)


def kernel(x_tokens, tw_row, fc_b2):
    L, B = x_tokens.shape
    V = tw_row.shape[1]
    assert V % 256 == 0 and B % _BB == 0 and L % 8 == 0

    tw_packed = _pack_table(tw_row)
    bias = fc_b2.reshape(1, 1)

    kern = functools.partial(_gather_kernel, inv_len=1.0 / L)
    out = pl.pallas_call(
        kern,
        out_shape=jax.ShapeDtypeStruct((8, B), jnp.float32),
        grid=(B // _BB,),
        in_specs=[
            pl.BlockSpec((L, _BB), lambda b: (0, b)),
            pl.BlockSpec(tw_packed.shape, lambda b: (0, 0)),
            pl.BlockSpec(memory_space=pltpu.MemorySpace.SMEM),
        ],
        out_specs=pl.BlockSpec((8, _BB), lambda b: (0, b)),
        compiler_params=pltpu.CompilerParams(
            dimension_semantics=("parallel",),
            vmem_limit_bytes=32 * 1024 * 1024,
        ),
    )(x_tokens.astype(jnp.int32), tw_packed, bias)

    return out[0, :]


# speedup vs baseline: 92.6493x; 1.9267x over previous
import functools

import jax
import jax.numpy as jnp
from jax import lax
from jax.experimental import pallas as pl
from jax.experimental.pallas import tpu as pltpu

_BB = 128
_CR = 200


def _gather_kernel(ids_ref,
                   tw_ref,
                   b_ref,
                   out_ref,
                   *, inv_len):
    L = ids_ref.shape[0]
    G = tw_ref.shape[0] // _CR
    n_rg = L // 8

    def body(g, acc):
        base = pl.multiple_of(g * _CR, 8)
        chunk = tw_ref[pl.ds(base, _CR), :]
        rows = [jnp.broadcast_to(chunk[rr:rr + 1, :], (8, _BB))
                for rr in range(_CR)]
        parts = []
        for rg in range(n_rg):
            ids = ids_ref[rg * 8:(rg + 1) * 8, :]
            lo = ids & 127
            pr = ids >> 8
            odd = (ids & 128) > 0
            a = acc[rg * 8:(rg + 1) * 8, :]
            for rr in range(_CR):
                pair = jnp.take_along_axis(rows[rr], lo, axis=1)
                v_even = pltpu.bitcast(pair << 16, jnp.float32)
                v_odd = pltpu.bitcast(pair & jnp.int32(-65536), jnp.float32)
                val = jnp.where(odd, v_odd, v_even)
                a = a + jnp.where(pr == g * _CR + rr, val, 0.0)
            parts.append(a)
        return jnp.concatenate(parts, axis=0)

    init = jnp.zeros((L, _BB), jnp.float32)
    if G == 1:
        total = body(0, init)
    else:
        total = lax.fori_loop(0, G, body, init)

    colsum = jnp.sum(total, axis=0, keepdims=True)
    logits = colsum * inv_len + b_ref[0, 0]
    out_ref[...] = jnp.broadcast_to(logits, out_ref.shape)


def _pack_table(tw_row):
    V = tw_row.shape[1]
    tw2 = tw_row.reshape(V // 128, 128)
    b16 = jax.lax.bitcast_convert_type(tw2.astype(jnp.bfloat16),
                                       jnp.uint16).astype(jnp.uint32)
    packed = (b16[0::2, :] | (b16[1::2, :] << 16)).astype(jnp.uint32)
    P = packed.shape[0]
    Pc = (P + _CR - 1) // _CR * _CR
    packed = jnp.pad(packed, ((0, Pc - P), (0, 0)))
    return jax.lax.bitcast_convert_type(packed, jnp.int32)


def kernel(x_tokens, tw_row, fc_b2):
    L, B = x_tokens.shape
    V = tw_row.shape[1]
    assert V % 256 == 0 and B % _BB == 0 and L % 8 == 0

    tw_packed = _pack_table(tw_row)
    bias = fc_b2.reshape(1, 1)

    kern = functools.partial(_gather_kernel, inv_len=1.0 / L)
    out = pl.pallas_call(
        kern,
        out_shape=jax.ShapeDtypeStruct((8, B), jnp.float32),
        grid=(B // _BB,),
        in_specs=[
            pl.BlockSpec((L, _BB), lambda b: (0, b)),
            pl.BlockSpec(tw_packed.shape, lambda b: (0, 0)),
            pl.BlockSpec(memory_space=pltpu.MemorySpace.SMEM),
        ],
        out_specs=pl.BlockSpec((8, _BB), lambda b: (0, b)),
        compiler_params=pltpu.CompilerParams(
            dimension_semantics=("parallel",),
            vmem_limit_bytes=32 * 1024 * 1024,
        ),
    )(x_tokens.astype(jnp.int32), tw_packed, bias)

    return out[0, :]


# speedup vs baseline: 94.7924x; 1.0231x over previous
import functools

import jax
import jax.numpy as jnp
from jax import lax
from jax.experimental import pallas as pl
from jax.experimental.pallas import tpu as pltpu

_BB = 128
_CR = 200


def _gather_kernel(ids_ref,
                   tw_ref,
                   b_ref,
                   out_ref,
                   *, inv_len):
    L = ids_ref.shape[0]
    G = tw_ref.shape[0] // _CR
    n_rg = L // 8

    del G
    chunk = tw_ref[...]
    rows = [jnp.broadcast_to(chunk[rr:rr + 1, :], (8, _BB))
            for rr in range(_CR)]

    total = jnp.zeros((8, _BB), jnp.float32)
    for rg in range(n_rg):
        ids = ids_ref[rg * 8:(rg + 1) * 8, :]
        lo = ids & 127
        pr = ids >> 8
        odd = (ids & 128) > 0
        a = jnp.zeros((8, _BB), jnp.int32)
        for rr in range(_CR):
            pair = jnp.take_along_axis(rows[rr], lo, axis=1)
            a = jnp.where(pr == rr, pair, a)
        v_even = pltpu.bitcast(a << 16, jnp.float32)
        v_odd = pltpu.bitcast(a & jnp.int32(-65536), jnp.float32)
        total = total + jnp.where(odd, v_odd, v_even)

    colsum = jnp.sum(total, axis=0, keepdims=True)
    logits = colsum * inv_len + b_ref[0, 0]
    out_ref[...] = jnp.broadcast_to(logits, out_ref.shape)


def _pack_table(tw_row):
    V = tw_row.shape[1]
    tw2 = tw_row.reshape(V // 128, 128)
    b16 = jax.lax.bitcast_convert_type(tw2.astype(jnp.bfloat16),
                                       jnp.uint16).astype(jnp.uint32)
    packed = (b16[0::2, :] | (b16[1::2, :] << 16)).astype(jnp.uint32)
    P = packed.shape[0]
    Pc = (P + _CR - 1) // _CR * _CR
    packed = jnp.pad(packed, ((0, Pc - P), (0, 0)))
    return jax.lax.bitcast_convert_type(packed, jnp.int32)


def kernel(x_tokens, tw_row, fc_b2):
    L, B = x_tokens.shape
    V = tw_row.shape[1]
    assert V % 256 == 0 and B % _BB == 0 and L % 8 == 0

    tw_packed = _pack_table(tw_row)
    bias = fc_b2.reshape(1, 1)

    kern = functools.partial(_gather_kernel, inv_len=1.0 / L)
    out = pl.pallas_call(
        kern,
        out_shape=jax.ShapeDtypeStruct((8, B), jnp.float32),
        grid=(B // _BB,),
        in_specs=[
            pl.BlockSpec((L, _BB), lambda b: (0, b)),
            pl.BlockSpec(tw_packed.shape, lambda b: (0, 0)),
            pl.BlockSpec(memory_space=pltpu.MemorySpace.SMEM),
        ],
        out_specs=pl.BlockSpec((8, _BB), lambda b: (0, b)),
        compiler_params=pltpu.CompilerParams(
            dimension_semantics=("parallel",),
            vmem_limit_bytes=32 * 1024 * 1024,
        ),
    )(x_tokens.astype(jnp.int32), tw_packed, bias)

    return out[0, :]


# speedup vs baseline: 96.6398x; 1.0195x over previous
import functools

import jax
import jax.numpy as jnp
from jax import lax
from jax.experimental import pallas as pl
from jax.experimental.pallas import tpu as pltpu

_BB = 128


def _gather_kernel(ids_ref,
                   tw_ref,
                   b_ref,
                   out_ref,
                   *, inv_len, n_real):
    L = ids_ref.shape[0]
    n_rg = L // 8

    chunk = tw_ref[...]
    rows = [jnp.broadcast_to(chunk[rr:rr + 1, :], (8, _BB))
            for rr in range(n_real)]

    total = jnp.zeros((8, _BB), jnp.float32)
    for rg in range(n_rg):
        ids = ids_ref[rg * 8:(rg + 1) * 8, :]
        lo = ids & 127
        pr = ids >> 8
        odd = (ids & 128) > 0
        a = jnp.zeros((8, _BB), jnp.int32)
        for rr in range(n_real):
            pair = jnp.take_along_axis(rows[rr], lo, axis=1)
            a = jnp.where(pr == rr, pair, a)
        v_even = pltpu.bitcast(a << 16, jnp.float32)
        v_odd = pltpu.bitcast(a & jnp.int32(-65536), jnp.float32)
        total = total + jnp.where(odd, v_odd, v_even)

    colsum = jnp.sum(total, axis=0, keepdims=True)
    logits = colsum * inv_len + b_ref[0, 0]
    out_ref[...] = jnp.broadcast_to(logits, out_ref.shape)


def _pack_table(tw_row):
    V = tw_row.shape[1]
    tw2 = tw_row.reshape(V // 128, 128)
    b16 = jax.lax.bitcast_convert_type(tw2.astype(jnp.bfloat16),
                                       jnp.uint16).astype(jnp.uint32)
    packed = (b16[0::2, :] | (b16[1::2, :] << 16)).astype(jnp.uint32)
    P = packed.shape[0]
    Pc = (P + 7) // 8 * 8
    packed = jnp.pad(packed, ((0, Pc - P), (0, 0)))
    return jax.lax.bitcast_convert_type(packed, jnp.int32)


def kernel(x_tokens, tw_row, fc_b2):
    L, B = x_tokens.shape
    V = tw_row.shape[1]
    assert V % 256 == 0 and B % _BB == 0 and L % 8 == 0

    tw_packed = _pack_table(tw_row)
    bias = fc_b2.reshape(1, 1)

    kern = functools.partial(_gather_kernel, inv_len=1.0 / L,
                             n_real=(V + 255) // 256)
    out = pl.pallas_call(
        kern,
        out_shape=jax.ShapeDtypeStruct((8, B), jnp.float32),
        grid=(B // _BB,),
        in_specs=[
            pl.BlockSpec((L, _BB), lambda b: (0, b)),
            pl.BlockSpec(tw_packed.shape, lambda b: (0, 0)),
            pl.BlockSpec(memory_space=pltpu.MemorySpace.SMEM),
        ],
        out_specs=pl.BlockSpec((8, _BB), lambda b: (0, b)),
        compiler_params=pltpu.CompilerParams(
            dimension_semantics=("parallel",),
            vmem_limit_bytes=32 * 1024 * 1024,
        ),
    )(x_tokens.astype(jnp.int32), tw_packed, bias)

    return out[0, :]


# speedup vs baseline: 97.6230x; 1.0102x over previous
import functools

import jax
import jax.numpy as jnp
from jax import lax
from jax.experimental import pallas as pl
from jax.experimental.pallas import tpu as pltpu

_BB = 256


def _gather_kernel(ids_ref,
                   tw_ref,
                   b_ref,
                   out_ref,
                   *, inv_len, n_real):
    L = ids_ref.shape[0]
    n_rg = L // 8

    chunk = tw_ref[...]
    rows = [jnp.broadcast_to(chunk[rr:rr + 1, :], (8, 128))
            for rr in range(n_real)]

    total = jnp.zeros((8, _BB), jnp.float32)
    for rg in range(n_rg):
        vals = []
        for h in range(_BB // 128):
            ids = ids_ref[rg * 8:(rg + 1) * 8,
                          h * 128:(h + 1) * 128]
            lo = ids & 127
            pr = ids >> 8
            odd = (ids & 128) > 0
            a = jnp.zeros((8, 128), jnp.int32)
            for rr in range(n_real):
                pair = jnp.take_along_axis(rows[rr], lo, axis=1)
                a = jnp.where(pr == rr, pair, a)
            v_even = pltpu.bitcast(a << 16, jnp.float32)
            v_odd = pltpu.bitcast(a & jnp.int32(-65536), jnp.float32)
            vals.append(jnp.where(odd, v_odd, v_even))
        total = total + (jnp.concatenate(vals, axis=1)
                         if len(vals) > 1 else vals[0])

    colsum = jnp.sum(total, axis=0, keepdims=True)
    logits = colsum * inv_len + b_ref[0, 0]
    out_ref[...] = jnp.broadcast_to(logits, out_ref.shape)


def _pack_table(tw_row):
    V = tw_row.shape[1]
    tw2 = tw_row.reshape(V // 128, 128)
    b16 = jax.lax.bitcast_convert_type(tw2.astype(jnp.bfloat16),
                                       jnp.uint16).astype(jnp.uint32)
    packed = (b16[0::2, :] | (b16[1::2, :] << 16)).astype(jnp.uint32)
    P = packed.shape[0]
    Pc = (P + 7) // 8 * 8
    packed = jnp.pad(packed, ((0, Pc - P), (0, 0)))
    return jax.lax.bitcast_convert_type(packed, jnp.int32)


def kernel(x_tokens, tw_row, fc_b2):
    L, B = x_tokens.shape
    V = tw_row.shape[1]
    assert V % 256 == 0 and B % _BB == 0 and L % 8 == 0

    tw_packed = _pack_table(tw_row)
    bias = fc_b2.reshape(1, 1)

    kern = functools.partial(_gather_kernel, inv_len=1.0 / L,
                             n_real=(V + 255) // 256)
    out = pl.pallas_call(
        kern,
        out_shape=jax.ShapeDtypeStruct((8, B), jnp.float32),
        grid=(B // _BB,),
        in_specs=[
            pl.BlockSpec((L, _BB), lambda b: (0, b)),
            pl.BlockSpec(tw_packed.shape, lambda b: (0, 0)),
            pl.BlockSpec(memory_space=pltpu.MemorySpace.SMEM),
        ],
        out_specs=pl.BlockSpec((8, _BB), lambda b: (0, b)),
        compiler_params=pltpu.CompilerParams(
            dimension_semantics=("parallel",),
            vmem_limit_bytes=32 * 1024 * 1024,
        ),
    )(x_tokens.astype(jnp.int32), tw_packed, bias)

    return out[0, :]


# speedup vs baseline: 97.6414x; 1.0002x over previous
import functools

import jax
import jax.numpy as jnp
from jax.experimental import pallas as pl
from jax.experimental.pallas import tpu as pltpu

_BB = 256


def _gather_kernel(ids_ref,
                   tw_ref,
                   b_ref,
                   out_ref,
                   *, inv_len, n_real):
    L = ids_ref.shape[0]
    n_rg = L // 8

    chunk = tw_ref[...]
    rows = [jnp.broadcast_to(chunk[rr:rr + 1, :], (8, 128))
            for rr in range(n_real)]

    total = jnp.zeros((8, _BB), jnp.float32)
    for rg in range(n_rg):
        vals = []
        for h in range(_BB // 128):
            ids = ids_ref[rg * 8:(rg + 1) * 8,
                          h * 128:(h + 1) * 128]
            lo = ids & 127
            pr = ids >> 8
            odd = (ids & 128) > 0
            a = jnp.zeros((8, 128), jnp.int32)
            for rr in range(n_real):
                pair = jnp.take_along_axis(rows[rr], lo, axis=1)
                a = jnp.where(pr == rr, pair, a)
            v_even = pltpu.bitcast(a << 16, jnp.float32)
            v_odd = pltpu.bitcast(a & jnp.int32(-65536), jnp.float32)
            vals.append(jnp.where(odd, v_odd, v_even))
        total = total + (jnp.concatenate(vals, axis=1)
                         if len(vals) > 1 else vals[0])

    colsum = jnp.sum(total, axis=0, keepdims=True)
    logits = colsum * inv_len + b_ref[0, 0]
    out_ref[...] = jnp.broadcast_to(logits, out_ref.shape)


def _pack_table(tw_row):
    V = tw_row.shape[1]
    tw2 = tw_row.reshape(V // 128, 128)
    b16 = jax.lax.bitcast_convert_type(tw2.astype(jnp.bfloat16),
                                       jnp.uint16).astype(jnp.uint32)
    packed = (b16[0::2, :] | (b16[1::2, :] << 16)).astype(jnp.uint32)
    P = packed.shape[0]
    Pc = (P + 7) // 8 * 8
    packed = jnp.pad(packed, ((0, Pc - P), (0, 0)))
    return jax.lax.bitcast_convert_type(packed, jnp.int32)


def kernel(x_tokens, tw_row, fc_b2):
    L, B = x_tokens.shape
    V = tw_row.shape[1]
    assert V % 256 == 0 and B % _BB == 0 and L % 8 == 0

    tw_packed = _pack_table(tw_row)
    bias = fc_b2.reshape(1, 1)

    kern = functools.partial(_gather_kernel, inv_len=1.0 / L,
                             n_real=(V + 255) // 256)
    out = pl.pallas_call(
        kern,
        out_shape=jax.ShapeDtypeStruct((8, B), jnp.float32),
        grid=(B // _BB,),
        in_specs=[
            pl.BlockSpec((L, _BB), lambda b: (0, b)),
            pl.BlockSpec(tw_packed.shape, lambda b: (0, 0)),
            pl.BlockSpec(memory_space=pltpu.MemorySpace.SMEM),
        ],
        out_specs=pl.BlockSpec((8, _BB), lambda b: (0, b)),
        compiler_params=pltpu.CompilerParams(
            dimension_semantics=("parallel",),
            vmem_limit_bytes=32 * 1024 * 1024,
        ),
    )(x_tokens.astype(jnp.int32), tw_packed, bias)

    return out[0, :]
